# Initial kernel scaffold; baseline (speedup 1.0000x reference)
import jax
import jax.numpy as jnp
from jax.experimental import pallas as pl


def kernel(q, k, v, eigs, lambda0, path_emb_weight, indices, path_type):
    raise NotImplementedError("write your pallas kernel here")


# speedup vs baseline: 2.6481x; 2.6481x over previous
import functools
import jax
import jax.numpy as jnp
from jax import lax
from jax.experimental import pallas as pl
from jax.experimental.pallas import tpu as pltpu
from jax.experimental.pallas import tpu_sc as plsc

N = 10000
D = 256
E = 160000
EIG = 16
L = 16
NC = 2
NS = 16
NW = NC * NS
EP = 163840
CPT = EP // NW
CB = 128
NCH = CPT // CB
GP = CB // L
NP_ = 10240
RPT = NP_ // NW
SPT = NP_ // NS
DH = D // 2
DX = 384
DHQ = 64
NPA = 10240
NPH = NPA // 2
SPH = NPH // NS
ZRH = 160
DU = 272
ZR = 160
KR = 80
NEG = -3.0e38

_mesh = plsc.VectorSubcoreMesh(core_axis_name="c", subcore_axis_name="s")


def _wid():
    return lax.axis_index("s") * NC + lax.axis_index("c")


@functools.partial(
    pl.kernel,
    mesh=_mesh,
    compiler_params=pltpu.CompilerParams(needs_layout_passes=False),
    out_type=(
        jax.ShapeDtypeStruct((EP,), jnp.float32),
        jax.ShapeDtypeStruct((EP,), jnp.float32),
        jax.ShapeDtypeStruct((NW * L,), jnp.float32),
        jax.ShapeDtypeStruct((NW * L,), jnp.float32),
    ),
    scratch_types=[
        pltpu.VMEM((CB,), jnp.int32),
        pltpu.VMEM((CB,), jnp.int32),
        pltpu.VMEM((CB,), jnp.int32),
        pltpu.VMEM((CB, DX), jnp.float32),
        pltpu.VMEM((CB, DX), jnp.float32),
        pltpu.VMEM((CB,), jnp.float32),
        pltpu.VMEM((CB,), jnp.float32),
        pltpu.VMEM((L,), jnp.float32),
        pltpu.VMEM((L,), jnp.float32),
        pltpu.SemaphoreType.DMA,
        pltpu.SemaphoreType.DMA,
    ],
)
def _k1(qx_h, kx_h, row_h, col_h, pt_h, pe_h,
        s0_h, s1_h, mx0_h, mx1_h,
        rowv, colv, ptv, qbuf, kbuf, s0buf, s1buf,
        pe_v, mxv, sem0, sem1):
    w = _wid()
    base = w * CPT
    pltpu.sync_copy(pe_h, pe_v)
    lanev = lax.broadcasted_iota(jnp.int32, (L,), 0)

    def chunk(t, carry):
        b = base + t * CB
        pltpu.sync_copy(row_h.at[pl.ds(b, CB)], rowv)
        pltpu.sync_copy(col_h.at[pl.ds(b, CB)], colv)
        pltpu.sync_copy(pt_h.at[pl.ds(b, CB)], ptv)
        cq = pltpu.async_copy(qx_h.at[rowv], qbuf, sem0)
        ck = pltpu.async_copy(kx_h.at[colv], kbuf, sem1)
        cq.wait()
        ck.wait()

        def group(g, c2):
            m0g, m1g = c2
            off = g * L
            ptg = ptv[pl.ds(off, L)]
            zv = plsc.load_gather(pe_v, [ptg])
            sv0 = jnp.full((L,), NEG, jnp.float32)
            for e16 in range(L):
                e = off + e16
                acc = qbuf[e, pl.ds(0, L)] * kbuf[e, pl.ds(0, L)]
                for j in range(1, DU // L):
                    acc = acc + qbuf[e, pl.ds(j * L, L)] * kbuf[e, pl.ds(j * L, L)]
                s0e = jnp.sum(acc)
                sv0 = jnp.where(lanev == e16, s0e, sv0)
            ge = b + off + lanev
            valid = ge < E
            sv0 = jnp.where(valid, sv0, jnp.float32(NEG))
            sv1 = jnp.where(valid, zv, jnp.float32(NEG))
            s0buf[pl.ds(off, L)] = sv0
            s1buf[pl.ds(off, L)] = sv1
            return (jnp.maximum(m0g, sv0), jnp.maximum(m1g, sv1))

        carry = lax.fori_loop(0, GP, group, carry)
        pltpu.sync_copy(s0buf, s0_h.at[pl.ds(b, CB)])
        pltpu.sync_copy(s1buf, s1_h.at[pl.ds(b, CB)])
        return carry

    neg = jnp.full((L,), NEG, jnp.float32)
    m0v, m1v = lax.fori_loop(0, NCH, chunk, (neg, neg))
    mxv[:] = m0v
    pltpu.sync_copy(mxv, mx0_h.at[pl.ds(w * L, L)])
    mxv[:] = m1v
    pltpu.sync_copy(mxv, mx1_h.at[pl.ds(w * L, L)])


@functools.partial(
    pl.kernel,
    mesh=_mesh,
    compiler_params=pltpu.CompilerParams(needs_layout_passes=False),
    out_type=(
        jax.ShapeDtypeStruct((NC * NP_,), jnp.float32),
        jax.ShapeDtypeStruct((NC * NP_,), jnp.float32),
    ),
    scratch_types=[
        pltpu.VMEM((CB,), jnp.int32),
        pltpu.VMEM((CB,), jnp.float32),
        pltpu.VMEM((CB,), jnp.float32),
        pltpu.VMEM((CB,), jnp.float32),
        pltpu.VMEM((CB,), jnp.float32),
        pltpu.VMEM((SPT,), jnp.float32),
        pltpu.VMEM((L,), jnp.float32),
        pltpu.VMEM((L,), jnp.float32),
        pltpu.VMEM_SHARED((NP_,), jnp.float32),
        pltpu.VMEM_SHARED((NP_,), jnp.float32),
    ],
)
def _k2(row_h, s0_h, s1_h, m0_h, m1_h, dp0_h, dp1_h,
        rowv, s0b, s1b, e0buf, e1buf, strip, m0r, m1r, d0s, d1s):
    w = _wid()
    sid = lax.axis_index("s")
    cid = lax.axis_index("c")
    base = w * CPT
    pltpu.sync_copy(m0_h, m0r)
    pltpu.sync_copy(m1_h, m1r)
    m0v = m0r[:]
    m1v = m1r[:]
    zl = jnp.zeros((L,), jnp.float32)

    def zb(i, _):
        strip[pl.ds(i * L, L)] = zl
        return 0

    lax.fori_loop(0, SPT // L, zb, 0)
    pltpu.sync_copy(strip, d0s.at[pl.ds(sid * SPT, SPT)])
    pltpu.sync_copy(strip, d1s.at[pl.ds(sid * SPT, SPT)])
    plsc.subcore_barrier()

    def chunk(t, _):
        b = base + t * CB
        pltpu.sync_copy(row_h.at[pl.ds(b, CB)], rowv)
        pltpu.sync_copy(s0_h.at[pl.ds(b, CB)], s0b)
        pltpu.sync_copy(s1_h.at[pl.ds(b, CB)], s1b)

        def group(g, _2):
            s = pl.ds(g * L, L)
            e0buf[s] = jnp.exp(s0b[s] - m0v)
            e1buf[s] = jnp.exp(s1b[s] - m1v)
            return 0

        lax.fori_loop(0, GP, group, 0)
        pltpu.sync_copy(e0buf, d0s.at[rowv], add=True)
        pltpu.sync_copy(e1buf, d1s.at[rowv], add=True)
        return 0

    lax.fori_loop(0, NCH, chunk, 0)
    plsc.subcore_barrier()
    pltpu.sync_copy(d0s.at[pl.ds(sid * SPT, SPT)], strip)
    pltpu.sync_copy(strip, dp0_h.at[pl.ds(cid * NP_ + sid * SPT, SPT)])
    pltpu.sync_copy(d1s.at[pl.ds(sid * SPT, SPT)], strip)
    pltpu.sync_copy(strip, dp1_h.at[pl.ds(cid * NP_ + sid * SPT, SPT)])


@functools.partial(
    pl.kernel,
    mesh=_mesh,
    compiler_params=pltpu.CompilerParams(needs_layout_passes=False),
    out_type=jax.ShapeDtypeStruct((4 * NC * NPH, DH), jnp.float32),
    scratch_types=[
        pltpu.VMEM((NP_,), jnp.float32),
        pltpu.VMEM((NP_,), jnp.float32),
        pltpu.VMEM((NP_,), jnp.float32),
        pltpu.VMEM((CB,), jnp.int32),
        pltpu.VMEM((CB,), jnp.int32),
        pltpu.VMEM((CB,), jnp.int32),
        pltpu.VMEM((CB,), jnp.float32),
        pltpu.VMEM((CB,), jnp.float32),
        pltpu.VMEM((CB, DH), jnp.float32),
        pltpu.VMEM((CB, DH), jnp.float32),
        pltpu.VMEM((ZRH, DH), jnp.float32),
        pltpu.VMEM((L,), jnp.float32),
        pltpu.VMEM((L,), jnp.float32),
        pltpu.VMEM_SHARED((NPH, DH), jnp.float32),
        pltpu.SemaphoreType.DMA,
    ],
)
def _k3(row_h, col_h, s0_h, s1_h, dp0_h, dp1_h, vh0_h, vh1_h, m0_h, m1_h,
        part_h,
        den0, den1, tmpd, rowv, colv, rwhalf, s0b, s1b, vbuf, contrib,
        zbuf, m0r, m1r, outp, sem):
    w = _wid()
    sid = lax.axis_index("s")
    cid = lax.axis_index("c")
    base = w * CPT
    pltpu.sync_copy(m0_h, m0r)
    pltpu.sync_copy(m1_h, m1r)
    m0v = m0r[:]
    m1v = m1r[:]
    pltpu.sync_copy(dp0_h.at[pl.ds(0, NP_)], den0)
    pltpu.sync_copy(dp0_h.at[pl.ds(NP_, NP_)], tmpd)

    def addd0(i, _):
        s = pl.ds(i * L, L)
        den0[s] = den0[s] + tmpd[s]
        return 0

    lax.fori_loop(0, NP_ // L, addd0, 0)
    pltpu.sync_copy(dp1_h.at[pl.ds(0, NP_)], den1)
    pltpu.sync_copy(dp1_h.at[pl.ds(NP_, NP_)], tmpd)

    def addd1(i, _):
        s = pl.ds(i * L, L)
        den1[s] = den1[s] + tmpd[s]
        return 0

    lax.fori_loop(0, NP_ // L, addd1, 0)

    zl = jnp.zeros((L,), jnp.float32)

    for ph in range(4):
        vh_h = (vh0_h, vh1_h)[ph // 2]
        qoff = (ph % 2) * DHQ

        def zb(i, _):
            r = i // (DH // L)
            c = (i % (DH // L)) * L
            zbuf[r, pl.ds(c, L)] = zl
            return 0

        lax.fori_loop(0, ZRH * (DH // L), zb, 0)

        def zs(i, _):
            pltpu.sync_copy(zbuf, outp.at[pl.ds(sid * SPH + i * ZRH, ZRH)])
            return 0

        lax.fori_loop(0, SPH // ZRH, zs, 0)
        plsc.subcore_barrier()

        def chunk(t, _):
            b = base + t * CB
            pltpu.sync_copy(row_h.at[pl.ds(b, CB)], rowv)
            pltpu.sync_copy(col_h.at[pl.ds(b, CB)], colv)
            pltpu.sync_copy(s0_h.at[pl.ds(b, CB)], s0b)
            pltpu.sync_copy(s1_h.at[pl.ds(b, CB)], s1b)
            pltpu.async_copy(vh_h.at[colv], vbuf, sem).wait()

            def group(g, _2):
                off = g * L
                sl = pl.ds(off, L)
                rw = rowv[sl]
                rwhalf[sl] = lax.shift_right_logical(rw, 1)
                pvec = jnp.bitwise_and(rw, 1)
                g0 = jnp.maximum(plsc.load_gather(den0, [rw]), jnp.float32(1e-30))
                g1 = jnp.maximum(plsc.load_gather(den1, [rw]), jnp.float32(1e-30))
                a0 = jnp.exp(s0b[sl] - m0v) / g0
                a1 = jnp.exp(s1b[sl] - m1v) / g1
                sv = jnp.float32(0.5) * (a0 + a1)
                for e16 in range(L):
                    se = sv[e16]
                    doff = pvec[e16] * DHQ
                    zoff = DHQ - doff
                    e = off + e16
                    for j in range(DHQ // L):
                        cs = j * L
                        contrib[e, pl.ds(doff + cs, L)] = (
                            vbuf[e, pl.ds(qoff + cs, L)] * se)
                        contrib[e, pl.ds(zoff + cs, L)] = zl
                return 0

            lax.fori_loop(0, GP, group, 0)
            pltpu.sync_copy(contrib, outp.at[rwhalf], add=True)
            return 0

        lax.fori_loop(0, NCH, chunk, 0)
        plsc.subcore_barrier()

        def fl(i, _):
            r0 = sid * SPH + i * ZRH
            pltpu.sync_copy(outp.at[pl.ds(r0, ZRH)], zbuf)
            pltpu.sync_copy(zbuf, part_h.at[pl.ds((ph * NC + cid) * NPH + r0, ZRH)])
            return 0

        lax.fori_loop(0, SPH // ZRH, fl, 0)


@functools.partial(
    pl.kernel,
    mesh=_mesh,
    compiler_params=pltpu.CompilerParams(needs_layout_passes=False),
    out_type=tuple(
        jax.ShapeDtypeStruct((NPH, DH), jnp.float32) for _ in range(4)
    ),
    scratch_types=[
        pltpu.VMEM((ZRH, DH), jnp.float32),
        pltpu.VMEM((ZRH, DH), jnp.float32),
    ],
)
def _k4(p_h, o0_h, o1_h, o2_h, o3_h, abuf, bbuf):
    w = _wid()
    base = w * (NPH // NW)

    for qi, dst in enumerate((o0_h, o1_h, o2_h, o3_h)):
        pltpu.sync_copy(p_h.at[pl.ds((2 * qi) * NPH + base, NPH // NW)], abuf)
        pltpu.sync_copy(p_h.at[pl.ds((2 * qi + 1) * NPH + base, NPH // NW)], bbuf)

        def add(jj, _2, qi=qi):
            rr = jj // (DH // L)
            cc = (jj % (DH // L)) * L
            sl = pl.ds(cc, L)
            abuf[rr, sl] = abuf[rr, sl] + bbuf[rr, sl]
            return 0

        lax.fori_loop(0, (NPH // NW) * (DH // L), add, 0)
        pltpu.sync_copy(abuf, dst.at[pl.ds(base, NPH // NW)])


def kernel(q, k, v, eigs, lambda0, path_emb_weight, indices, path_type):
    row = indices[0]
    col = indices[1]
    pad = EP - E
    rowp = jnp.concatenate([row, jnp.zeros((pad,), jnp.int32)])
    colp = jnp.concatenate([col, jnp.zeros((pad,), jnp.int32)])
    ptp = jnp.concatenate([path_type, jnp.zeros((pad,), jnp.int32)])
    npth = path_emb_weight.shape[0]
    pe16 = jnp.zeros((L,), jnp.float32).at[:npth].set(path_emb_weight[:, 0])
    el = jnp.exp(lambda0[0])
    zpad = jnp.zeros((q.shape[0], DX - D - EIG), jnp.float32)
    qx = jnp.concatenate([q * jnp.float32(0.0625), el * eigs, zpad], axis=1)
    kx = jnp.concatenate([k, eigs, zpad], axis=1)
    s0, s1, mx0, mx1 = _k1(qx, kx, rowp, colp, ptp, pe16)
    m0 = jnp.full((L,), jnp.max(mx0), jnp.float32)
    m1 = jnp.full((L,), jnp.max(mx1), jnp.float32)
    dp0, dp1 = _k2(rowp, s0, s1, m0, m1)
    part = _k3(rowp, colp, s0, s1, dp0, dp1, v[:, :DH], v[:, DH:], m0, m1)
    o0, o1, o2, o3 = _k4(part)
    qs = [o.reshape(NPA, DHQ)[:N] for o in (o0, o1, o2, o3)]
    return jnp.concatenate(qs, axis=1)
